# baseline (speedup 1.0000x reference)
import functools

import jax
import jax.numpy as jnp
from jax import lax
from jax.experimental import pallas as pl
from jax.experimental.pallas import tpu as pltpu
from jax.experimental.pallas import tpu_sc as plsc

B, L, K, H = 8, 2048, 32, 128
SCALE = 30.0
R = 512
E_BLK = R * K
TOTAL = B * L * K

HP = H // 2
NC, NS = 2, 16
NW = NC * NS
PER_W = TOTAL // NW
CH = 128
SUP = 256
GP = SUP // CH
NSUP = PER_W // SUP


def _gelu(x):
    return 0.5 * x * (1.0 + lax.erf(x * (2.0 ** -0.5)))


def _gelu2_bf(x):
    return x * (jnp.bfloat16(1.0) + lax.erf(x * jnp.bfloat16(2.0 ** -0.5)))


def _ln(x, g, b, eps=1e-5):
    m = jnp.mean(x, axis=-1, keepdims=True)
    c = x - m
    v = jnp.mean(c * c, axis=-1, keepdims=True)
    return c * lax.rsqrt(v + eps) * g + b


def _sc_gather(table, idx_flat, nb):
    total = nb * L * K
    per_w = total // NW
    nsup = per_w // SUP

    def body(table_hbm, idx_hbm, out_hbm,
             ib0, ib1, rb0, rb1, si0, si1, so0, so1, sg0, sg1):
        wid = lax.axis_index("s") * NC + lax.axis_index("c")
        base = wid * per_w
        ibs, rbs, sis = (ib0, ib1), (rb0, rb1), (si0, si1)
        sos, sgs = (so0, so1), (sg0, sg1)

        for b in range(2):
            pltpu.async_copy(idx_hbm.at[pl.ds(base + b * SUP, SUP)],
                             ibs[b], sis[b])

        def fire_gathers(s2, b):
            s = s2 * 2 + b
            off = base + s * SUP
            pltpu.make_async_copy(idx_hbm.at[pl.ds(off, SUP)],
                                  ibs[b], sis[b]).wait()

            @pl.when(s2 > 0)
            def _wait_prev_out():
                pltpu.make_async_copy(
                    rbs[b], out_hbm.at[pl.ds(off - 2 * SUP, SUP)],
                    sos[b]).wait()

            for j in range(GP):
                pltpu.async_copy(table_hbm.at[ibs[b].at[pl.ds(j * CH, CH)]],
                                 rbs[b].at[pl.ds(j * CH, CH)], sgs[b])

        def drain_and_emit(s2, b):
            s = s2 * 2 + b
            off = base + s * SUP
            for j in range(GP):
                pltpu.make_async_copy(
                    table_hbm.at[ibs[b].at[pl.ds(j * CH, CH)]],
                    rbs[b].at[pl.ds(j * CH, CH)], sgs[b]).wait()

            @pl.when(s2 < nsup // 2 - 1)
            def _prefetch_idx():
                pltpu.async_copy(idx_hbm.at[pl.ds(off + 2 * SUP, SUP)],
                                 ibs[b], sis[b])

            pltpu.async_copy(rbs[b], out_hbm.at[pl.ds(off, SUP)], sos[b])

        def outer(s2, carry):
            fire_gathers(s2, 0)
            fire_gathers(s2, 1)
            drain_and_emit(s2, 0)
            drain_and_emit(s2, 1)
            return carry

        lax.fori_loop(0, nsup // 2, outer, 0)

        for b in range(2):
            off = base + (nsup - 2 + b) * SUP
            pltpu.make_async_copy(rbs[b], out_hbm.at[pl.ds(off, SUP)],
                                  sos[b]).wait()

    mesh = plsc.VectorSubcoreMesh(core_axis_name="c", subcore_axis_name="s",
                                  num_cores=NC, num_subcores=NS)
    out = pl.kernel(
        body,
        out_type=jax.ShapeDtypeStruct((total, H), jnp.float32),
        mesh=mesh,
        scratch_types=[
            pltpu.VMEM((SUP,), jnp.int32),
            pltpu.VMEM((SUP,), jnp.int32),
            pltpu.VMEM((SUP, H), jnp.float32),
            pltpu.VMEM((SUP, H), jnp.float32),
            pltpu.SemaphoreType.DMA,
            pltpu.SemaphoreType.DMA,
            pltpu.SemaphoreType.DMA,
            pltpu.SemaphoreType.DMA,
            pltpu.SemaphoreType.DMA,
            pltpu.SemaphoreType.DMA,
        ],
        name="sc_neighbor_gather",
    )(table, idx_flat)
    return out.reshape(nb, L * K, H)


def _node_body(hv_ref, he_ref, g1_ref, ma_ref, mv_ref,
               w1s_ref, w1en_ref, b1_ref, w2_ref, b2_ref, w3_ref, b3_ref,
               n1g_ref, n1b_ref, win_ref, bin_ref, wout_ref, bout_ref,
               n2g_ref, n2b_ref,
               out_ref, outb_ref):
    f32 = jnp.float32
    bf = jnp.bfloat16
    hv = hv_ref[0]
    hvb = hv.astype(bf)
    pre_s = jnp.dot(hvb, w1s_ref[...], preferred_element_type=f32) + b1_ref[...]
    psb = pre_s.astype(bf)
    he = he_ref[0].astype(bf)
    g1 = g1_ref[0].astype(bf)
    x = jnp.concatenate([he, g1], axis=1)
    t = jnp.dot(x, w1en_ref[...], preferred_element_type=f32).astype(bf)
    t = t.reshape(R, K, H) + psb[:, None, :]
    t = _gelu2_bf(t).reshape(E_BLK, H)
    t = (jnp.dot(t, w2_ref[...], preferred_element_type=f32).astype(bf)
         + b2_ref[...].astype(bf))
    t = _gelu2_bf(t)
    xs = jnp.sum(t.reshape(R, K, H) * ma_ref[0][:, :, None], axis=1)
    msum = jnp.sum(ma_ref[0], axis=1, keepdims=True)
    dh = (jnp.dot(xs.astype(bf), w3_ref[...], preferred_element_type=f32)
          + msum * b3_ref[...]) * (1.0 / SCALE)
    h1 = _ln(hv + dh, n1g_ref[...], n1b_ref[...])
    ff = (jnp.dot(h1.astype(bf), win_ref[...],
                  preferred_element_type=f32).astype(bf)
          + bin_ref[...].astype(bf))
    ff = _gelu2_bf(ff)
    d2 = jnp.dot(ff, wout_ref[...], preferred_element_type=f32) + bout_ref[...]
    h2 = _ln(h1 + d2, n2g_ref[...], n2b_ref[...]) * mv_ref[0]
    out_ref[0] = h2
    outb_ref[0] = h2.astype(bf)


def _node_update(h_V, h_E2, g1, mask_attend, mask_V3, wp, nb):
    grid = (nb, L // R)
    full = lambda shape: pl.BlockSpec(shape, lambda b, i: (0,) * len(shape))
    in_specs = [
        pl.BlockSpec((1, R, H), lambda b, i: (b, i, 0)),
        pl.BlockSpec((1, E_BLK, H), lambda b, i: (b, i, 0)),
        pl.BlockSpec((1, E_BLK, H), lambda b, i: (b, i, 0)),
        pl.BlockSpec((1, R, K), lambda b, i: (b, i, 0)),
        pl.BlockSpec((1, R, 1), lambda b, i: (b, i, 0)),
        full((H, H)), full((2 * H, H)), full((1, H)),
        full((H, H)), full((1, H)), full((H, H)), full((1, H)),
        full((1, H)), full((1, H)),
        full((H, 4 * H)), full((1, 4 * H)), full((4 * H, H)), full((1, H)),
        full((1, H)), full((1, H)),
    ]
    out_specs = [
        pl.BlockSpec((1, R, H), lambda b, i: (b, i, 0)),
        pl.BlockSpec((1, R, H), lambda b, i: (b, i, 0)),
    ]
    return pl.pallas_call(
        _node_body,
        grid=grid,
        in_specs=in_specs,
        out_specs=out_specs,
        out_shape=[
            jax.ShapeDtypeStruct((nb, L, H), jnp.float32),
            jax.ShapeDtypeStruct((nb, L, H), jnp.bfloat16),
        ],
        name="tc_node_update",
    )(h_V, h_E2, g1, mask_attend, mask_V3, *wp)


def _edge_body(he_ref, g2_ref, hv2_ref,
               w1s_ref, w1en_ref, b1_ref, w2_ref, b2_ref, w3_ref, b3_ref,
               n3g_ref, n3b_ref, out_ref):
    f32 = jnp.float32
    bf = jnp.bfloat16
    hv2 = hv2_ref[0]
    pre_s = jnp.dot(hv2, w1s_ref[...], preferred_element_type=f32) + b1_ref[...]
    psb = pre_s.astype(bf)
    heb = he_ref[0].astype(bf)
    x = jnp.concatenate([heb, g2_ref[0].astype(bf)], axis=1)
    t = jnp.dot(x, w1en_ref[...], preferred_element_type=f32).astype(bf)
    t = t.reshape(R, K, H) + psb[:, None, :]
    t = _gelu2_bf(t).reshape(E_BLK, H)
    t = (jnp.dot(t, w2_ref[...], preferred_element_type=f32).astype(bf)
         + b2_ref[...].astype(bf))
    t = _gelu2_bf(t)
    msg = (jnp.dot(t, w3_ref[...], preferred_element_type=f32).astype(bf)
           + b3_ref[...].astype(bf))
    u = heb + msg
    m = jnp.mean(u, axis=-1, keepdims=True, dtype=f32)
    s2 = jnp.mean(u * u, axis=-1, keepdims=True, dtype=f32)
    r = lax.rsqrt(jnp.maximum(s2 - m * m, 0.0) + 1e-5)
    n = (u - m.astype(bf)) * r.astype(bf)
    out_ref[0] = (n * n3g_ref[...].astype(bf)
                  + n3b_ref[...].astype(bf)).astype(f32)


def _edge_update(h_E2, g2, hV2b, wp):
    grid = (B, L // R)
    full = lambda shape: pl.BlockSpec(shape, lambda b, i: (0,) * len(shape))
    in_specs = [
        pl.BlockSpec((1, E_BLK, H), lambda b, i: (b, i, 0)),
        pl.BlockSpec((1, E_BLK, H), lambda b, i: (b, i, 0)),
        pl.BlockSpec((1, R, H), lambda b, i: (b, i, 0)),
        full((H, H)), full((2 * H, H)), full((1, H)),
        full((H, H)), full((1, H)), full((H, H)), full((1, H)),
        full((1, H)), full((1, H)),
    ]
    return pl.pallas_call(
        _edge_body,
        grid=grid,
        in_specs=in_specs,
        out_specs=pl.BlockSpec((1, E_BLK, H), lambda b, i: (b, i, 0)),
        out_shape=jax.ShapeDtypeStruct((B, L * K, H), jnp.float32),
        compiler_params=pltpu.CompilerParams(
            vmem_limit_bytes=63 * 1024 * 1024),
        name="tc_edge_update",
    )(h_E2, g2, hV2b, *wp)


def kernel(h_V, h_E, E_idx, mask_V, mask_attend, params):
    p = params
    bf = jnp.bfloat16
    f32 = jnp.float32

    h_E2 = h_E.reshape(B, L * K, H)
    idx_flat = (E_idx.astype(jnp.int32)
                + (jnp.arange(B, dtype=jnp.int32) * L)[:, None, None])
    idx_flat = idx_flat.reshape(TOTAL)
    mask_V3 = mask_V.reshape(B, L, 1)

    def wt(w):
        return jnp.transpose(w).astype(bf)

    def bias(b, n):
        return b.reshape(1, n).astype(f32)

    def wth(w):
        return jnp.transpose(0.5 * w).astype(bf)

    w1 = jnp.transpose(p['W1_w']).astype(bf)
    wp_node = (
        w1[:H], w1[H:], bias(p['W1_b'], H),
        wth(p['W2_w']), bias(p['W2_b'], H),
        wth(p['W3_w']), bias(p['W3_b'], H),
        bias(p['n1_g'], H), bias(p['n1_b'], H),
        wt(p['Win_w']), bias(p['Win_b'], 4 * H),
        wth(p['Wout_w']), bias(p['Wout_b'], H),
        bias(p['n2_g'], H), bias(p['n2_b'], H),
    )
    w11 = jnp.transpose(p['W11_w']).astype(bf)
    wp_edge = (
        w11[:H], w11[H:], bias(p['W11_b'], H),
        wth(p['W12_w']), bias(p['W12_b'], H),
        wth(p['W13_w']), bias(p['W13_b'], H),
        bias(p['n3_g'], H), bias(p['n3_b'], H),
    )

    g1 = _sc_gather(h_V.reshape(B * L, H), idx_flat, B)
    hV_new, hV_new_b = _node_update(
        h_V, h_E2, g1, mask_attend, mask_V3, wp_node, B)
    g2 = _sc_gather(hV_new.reshape(B * L, H), idx_flat, B)
    hE_new = _edge_update(h_E2, g2, hV_new_b, wp_edge)
    return (hV_new, hE_new.reshape(B, L, K, H))


# speedup vs baseline: 1.0170x; 1.0170x over previous
import functools

import jax
import jax.numpy as jnp
from jax import lax
from jax.experimental import pallas as pl
from jax.experimental.pallas import tpu as pltpu
from jax.experimental.pallas import tpu_sc as plsc

B, L, K, H = 8, 2048, 32, 128
SCALE = 30.0
R = 512
E_BLK = R * K
TOTAL = B * L * K

HP = H // 2
NC, NS = 2, 16
NW = NC * NS
PER_W = TOTAL // NW
CH = 128
SUP = 256
GP = SUP // CH
NSUP = PER_W // SUP


def _gelu(x):
    return 0.5 * x * (1.0 + lax.erf(x * (2.0 ** -0.5)))


def _gelu_bf(x):
    h = jnp.bfloat16(0.5)
    return h * x * (jnp.bfloat16(1.0) + lax.erf(x * jnp.bfloat16(2.0 ** -0.5)))


def _ln(x, g, b, eps=1e-5):
    m = jnp.mean(x, axis=-1, keepdims=True)
    c = x - m
    v = jnp.mean(c * c, axis=-1, keepdims=True)
    return c * lax.rsqrt(v + eps) * g + b


def _sc_gather(table, idx_flat, nb):
    total = nb * L * K
    per_w = total // NW
    nsup = per_w // SUP

    def body(table_hbm, idx_hbm, out_hbm,
             ib0, ib1, rb0, rb1, si0, si1, so0, so1, sg):
        wid = lax.axis_index("s") * NC + lax.axis_index("c")
        base = wid * per_w
        ibs, rbs, sis, sos = (ib0, ib1), (rb0, rb1), (si0, si1), (so0, so1)

        for b in range(2):
            pltpu.async_copy(idx_hbm.at[pl.ds(base + b * SUP, SUP)],
                             ibs[b], sis[b])

        def outer(s2, carry):
            for b in range(2):
                s = s2 * 2 + b
                off = base + s * SUP
                pltpu.make_async_copy(idx_hbm.at[pl.ds(off, SUP)],
                                      ibs[b], sis[b]).wait()

                @pl.when(s2 > 0)
                def _wait_prev_out():
                    pltpu.make_async_copy(
                        rbs[b], out_hbm.at[pl.ds(off - 2 * SUP, SUP)],
                        sos[b]).wait()

                cps = [pltpu.async_copy(
                           table_hbm.at[ibs[b].at[pl.ds(j * CH, CH)]],
                           rbs[b].at[pl.ds(j * CH, CH)], sg)
                       for j in range(GP)]
                for c in cps:
                    c.wait()

                @pl.when(s2 < nsup // 2 - 1)
                def _prefetch_idx():
                    pltpu.async_copy(idx_hbm.at[pl.ds(off + 2 * SUP, SUP)],
                                     ibs[b], sis[b])

                pltpu.async_copy(rbs[b], out_hbm.at[pl.ds(off, SUP)], sos[b])
            return carry

        lax.fori_loop(0, nsup // 2, outer, 0)

        for b in range(2):
            off = base + (nsup - 2 + b) * SUP
            pltpu.make_async_copy(rbs[b], out_hbm.at[pl.ds(off, SUP)],
                                  sos[b]).wait()

    mesh = plsc.VectorSubcoreMesh(core_axis_name="c", subcore_axis_name="s",
                                  num_cores=NC, num_subcores=NS)
    out = pl.kernel(
        body,
        out_type=jax.ShapeDtypeStruct((total, H), jnp.float32),
        mesh=mesh,
        scratch_types=[
            pltpu.VMEM((SUP,), jnp.int32),
            pltpu.VMEM((SUP,), jnp.int32),
            pltpu.VMEM((SUP, H), jnp.float32),
            pltpu.VMEM((SUP, H), jnp.float32),
            pltpu.SemaphoreType.DMA,
            pltpu.SemaphoreType.DMA,
            pltpu.SemaphoreType.DMA,
            pltpu.SemaphoreType.DMA,
            pltpu.SemaphoreType.DMA,
        ],
        name="sc_neighbor_gather",
    )(table, idx_flat)
    return out.reshape(nb, L * K, H)


def _node_body(hv_ref, he_ref, g1_ref, ma_ref, mv_ref,
               w1s_ref, w1en_ref, b1_ref, w2_ref, b2_ref, w3_ref, b3_ref,
               n1g_ref, n1b_ref, win_ref, bin_ref, wout_ref, bout_ref,
               n2g_ref, n2b_ref,
               out_ref, outb_ref):
    f32 = jnp.float32
    bf = jnp.bfloat16
    hv = hv_ref[0]
    hvb = hv.astype(bf)
    pre_s = jnp.dot(hvb, w1s_ref[...], preferred_element_type=f32) + b1_ref[...]
    psb = pre_s.astype(bf)
    he = he_ref[0].astype(bf)
    g1 = g1_ref[0].astype(bf)
    x = jnp.concatenate([he, g1], axis=1)
    t = jnp.dot(x, w1en_ref[...], preferred_element_type=f32).astype(bf)
    t = t.reshape(R, K, H) + psb[:, None, :]
    t = _gelu_bf(t).reshape(E_BLK, H)
    t = (jnp.dot(t, w2_ref[...], preferred_element_type=f32).astype(bf)
         + b2_ref[...].astype(bf))
    t = _gelu_bf(t)
    xs = jnp.sum(t.reshape(R, K, H) * ma_ref[0][:, :, None], axis=1)
    msum = jnp.sum(ma_ref[0], axis=1, keepdims=True)
    dh = (jnp.dot(xs.astype(bf), w3_ref[...], preferred_element_type=f32)
          + msum * b3_ref[...]) * (1.0 / SCALE)
    h1 = _ln(hv + dh, n1g_ref[...], n1b_ref[...])
    ff = (jnp.dot(h1.astype(bf), win_ref[...],
                  preferred_element_type=f32).astype(bf)
          + bin_ref[...].astype(bf))
    ff = _gelu_bf(ff)
    d2 = jnp.dot(ff, wout_ref[...], preferred_element_type=f32) + bout_ref[...]
    h2 = _ln(h1 + d2, n2g_ref[...], n2b_ref[...]) * mv_ref[0]
    out_ref[0] = h2
    outb_ref[0] = h2.astype(bf)


def _node_update(h_V, h_E2, g1, mask_attend, mask_V3, wp, nb):
    grid = (nb, L // R)
    full = lambda shape: pl.BlockSpec(shape, lambda b, i: (0,) * len(shape))
    in_specs = [
        pl.BlockSpec((1, R, H), lambda b, i: (b, i, 0)),
        pl.BlockSpec((1, E_BLK, H), lambda b, i: (b, i, 0)),
        pl.BlockSpec((1, E_BLK, H), lambda b, i: (b, i, 0)),
        pl.BlockSpec((1, R, K), lambda b, i: (b, i, 0)),
        pl.BlockSpec((1, R, 1), lambda b, i: (b, i, 0)),
        full((H, H)), full((2 * H, H)), full((1, H)),
        full((H, H)), full((1, H)), full((H, H)), full((1, H)),
        full((1, H)), full((1, H)),
        full((H, 4 * H)), full((1, 4 * H)), full((4 * H, H)), full((1, H)),
        full((1, H)), full((1, H)),
    ]
    out_specs = [
        pl.BlockSpec((1, R, H), lambda b, i: (b, i, 0)),
        pl.BlockSpec((1, R, H), lambda b, i: (b, i, 0)),
    ]
    return pl.pallas_call(
        _node_body,
        grid=grid,
        in_specs=in_specs,
        out_specs=out_specs,
        out_shape=[
            jax.ShapeDtypeStruct((nb, L, H), jnp.float32),
            jax.ShapeDtypeStruct((nb, L, H), jnp.bfloat16),
        ],
        name="tc_node_update",
    )(h_V, h_E2, g1, mask_attend, mask_V3, *wp)


def _edge_body(he_ref, g2_ref, hv2_ref,
               w1s_ref, w1en_ref, b1_ref, w2_ref, b2_ref, w3_ref, b3_ref,
               n3g_ref, n3b_ref, out_ref):
    f32 = jnp.float32
    bf = jnp.bfloat16
    hv2 = hv2_ref[0]
    pre_s = jnp.dot(hv2, w1s_ref[...], preferred_element_type=f32) + b1_ref[...]
    psb = pre_s.astype(bf)
    heb = he_ref[0].astype(bf)
    x = jnp.concatenate([heb, g2_ref[0].astype(bf)], axis=1)
    t = jnp.dot(x, w1en_ref[...], preferred_element_type=f32).astype(bf)
    t = t.reshape(R, K, H) + psb[:, None, :]
    t = _gelu_bf(t).reshape(E_BLK, H)
    t = (jnp.dot(t, w2_ref[...], preferred_element_type=f32).astype(bf)
         + b2_ref[...].astype(bf))
    t = _gelu_bf(t)
    msg = (jnp.dot(t, w3_ref[...], preferred_element_type=f32).astype(bf)
           + b3_ref[...].astype(bf))
    u = heb + msg
    m = jnp.mean(u, axis=-1, keepdims=True)
    c = u - m
    v = jnp.mean(c * c, axis=-1, keepdims=True)
    n = c * lax.rsqrt(v + jnp.bfloat16(1e-5))
    out_ref[0] = (n * n3g_ref[...].astype(bf)
                  + n3b_ref[...].astype(bf)).astype(f32)


def _edge_update(h_E2, g2, hV2b, wp):
    grid = (B, L // R)
    full = lambda shape: pl.BlockSpec(shape, lambda b, i: (0,) * len(shape))
    in_specs = [
        pl.BlockSpec((1, E_BLK, H), lambda b, i: (b, i, 0)),
        pl.BlockSpec((1, E_BLK, H), lambda b, i: (b, i, 0)),
        pl.BlockSpec((1, R, H), lambda b, i: (b, i, 0)),
        full((H, H)), full((2 * H, H)), full((1, H)),
        full((H, H)), full((1, H)), full((H, H)), full((1, H)),
        full((1, H)), full((1, H)),
    ]
    return pl.pallas_call(
        _edge_body,
        grid=grid,
        in_specs=in_specs,
        out_specs=pl.BlockSpec((1, E_BLK, H), lambda b, i: (b, i, 0)),
        out_shape=jax.ShapeDtypeStruct((B, L * K, H), jnp.float32),
        compiler_params=pltpu.CompilerParams(
            vmem_limit_bytes=63 * 1024 * 1024),
        name="tc_edge_update",
    )(h_E2, g2, hV2b, *wp)


def kernel(h_V, h_E, E_idx, mask_V, mask_attend, params):
    p = params
    bf = jnp.bfloat16
    f32 = jnp.float32

    h_E2 = h_E.reshape(B, L * K, H)
    idx_flat = (E_idx.astype(jnp.int32)
                + (jnp.arange(B, dtype=jnp.int32) * L)[:, None, None])
    idx_flat = idx_flat.reshape(TOTAL)
    mask_V3 = mask_V.reshape(B, L, 1)

    def wt(w):
        return jnp.transpose(w).astype(bf)

    def bias(b, n):
        return b.reshape(1, n).astype(f32)

    w1 = jnp.transpose(p['W1_w']).astype(bf)
    wp_node = (
        w1[:H], w1[H:], bias(p['W1_b'], H),
        wt(p['W2_w']), bias(p['W2_b'], H),
        wt(p['W3_w']), bias(p['W3_b'], H),
        bias(p['n1_g'], H), bias(p['n1_b'], H),
        wt(p['Win_w']), bias(p['Win_b'], 4 * H),
        wt(p['Wout_w']), bias(p['Wout_b'], H),
        bias(p['n2_g'], H), bias(p['n2_b'], H),
    )
    w11 = jnp.transpose(p['W11_w']).astype(bf)
    wp_edge = (
        w11[:H], w11[H:], bias(p['W11_b'], H),
        wt(p['W12_w']), bias(p['W12_b'], H),
        wt(p['W13_w']), bias(p['W13_b'], H),
        bias(p['n3_g'], H), bias(p['n3_b'], H),
    )

    g1 = _sc_gather(h_V.reshape(B * L, H), idx_flat, B)
    hV_new, hV_new_b = _node_update(
        h_V, h_E2, g1, mask_attend, mask_V3, wp_node, B)
    g2 = _sc_gather(hV_new.reshape(B * L, H), idx_flat, B)
    hE_new = _edge_update(h_E2, g2, hV_new_b, wp_edge)
    return (hV_new, hE_new.reshape(B, L, K, H))


# speedup vs baseline: 1.0170x; 1.0000x over previous
import jax
import jax.numpy as jnp
from jax import lax
from jax.experimental import pallas as pl
from jax.experimental.pallas import tpu as pltpu
from jax.experimental.pallas import tpu_sc as plsc

B, L, K, H = 8, 2048, 32, 128
SCALE = 30.0
R = 512
E_BLK = R * K
TOTAL = B * L * K

NC, NS = 2, 16
NW = NC * NS
CH = 128
SUP = 256
GP = SUP // CH


def _gelu_bf(x):
    h = jnp.bfloat16(0.5)
    return h * x * (jnp.bfloat16(1.0) + lax.erf(x * jnp.bfloat16(2.0 ** -0.5)))


def _ln(x, g, b, eps=1e-5):
    m = jnp.mean(x, axis=-1, keepdims=True)
    c = x - m
    v = jnp.mean(c * c, axis=-1, keepdims=True)
    return c * lax.rsqrt(v + eps) * g + b


def _sc_gather(table, idx_flat, nb):
    total = nb * L * K
    per_w = total // NW
    nsup = per_w // SUP

    def body(table_hbm, idx_hbm, out_hbm,
             ib0, ib1, rb0, rb1, si0, si1, so0, so1, sg):
        wid = lax.axis_index("s") * NC + lax.axis_index("c")
        base = wid * per_w
        ibs, rbs, sis, sos = (ib0, ib1), (rb0, rb1), (si0, si1), (so0, so1)

        for b in range(2):
            pltpu.async_copy(idx_hbm.at[pl.ds(base + b * SUP, SUP)],
                             ibs[b], sis[b])

        def outer(s2, carry):
            for b in range(2):
                s = s2 * 2 + b
                off = base + s * SUP
                pltpu.make_async_copy(idx_hbm.at[pl.ds(off, SUP)],
                                      ibs[b], sis[b]).wait()

                @pl.when(s2 > 0)
                def _wait_prev_out():
                    pltpu.make_async_copy(
                        rbs[b], out_hbm.at[pl.ds(off - 2 * SUP, SUP)],
                        sos[b]).wait()

                cps = [pltpu.async_copy(
                           table_hbm.at[ibs[b].at[pl.ds(j * CH, CH)]],
                           rbs[b].at[pl.ds(j * CH, CH)], sg)
                       for j in range(GP)]
                for c in cps:
                    c.wait()

                @pl.when(s2 < nsup // 2 - 1)
                def _prefetch_idx():
                    pltpu.async_copy(idx_hbm.at[pl.ds(off + 2 * SUP, SUP)],
                                     ibs[b], sis[b])

                pltpu.async_copy(rbs[b], out_hbm.at[pl.ds(off, SUP)], sos[b])
            return carry

        lax.fori_loop(0, nsup // 2, outer, 0)

        for b in range(2):
            off = base + (nsup - 2 + b) * SUP
            pltpu.make_async_copy(rbs[b], out_hbm.at[pl.ds(off, SUP)],
                                  sos[b]).wait()

    mesh = plsc.VectorSubcoreMesh(core_axis_name="c", subcore_axis_name="s",
                                  num_cores=NC, num_subcores=NS)
    out = pl.kernel(
        body,
        out_type=jax.ShapeDtypeStruct((total, H), jnp.float32),
        mesh=mesh,
        scratch_types=[
            pltpu.VMEM((SUP,), jnp.int32),
            pltpu.VMEM((SUP,), jnp.int32),
            pltpu.VMEM((SUP, H), jnp.float32),
            pltpu.VMEM((SUP, H), jnp.float32),
            pltpu.SemaphoreType.DMA,
            pltpu.SemaphoreType.DMA,
            pltpu.SemaphoreType.DMA,
            pltpu.SemaphoreType.DMA,
            pltpu.SemaphoreType.DMA,
        ],
        name="sc_neighbor_gather",
    )(table, idx_flat)
    return out.reshape(nb, L * K, H)


def _node_body(hv_ref, he_ref, g1_ref, ma_ref, mv_ref,
               w1s_ref, w1en_ref, b1_ref, w2_ref, b2_ref, w3_ref, b3_ref,
               n1g_ref, n1b_ref, win_ref, bin_ref, wout_ref, bout_ref,
               n2g_ref, n2b_ref,
               out_ref, outb_ref):
    f32 = jnp.float32
    bf = jnp.bfloat16
    hv = hv_ref[0]
    hvb = hv.astype(bf)
    pre_s = jnp.dot(hvb, w1s_ref[...], preferred_element_type=f32) + b1_ref[...]
    psb = pre_s.astype(bf)
    he = he_ref[0].astype(bf)
    g1 = g1_ref[0].astype(bf)
    x = jnp.concatenate([he, g1], axis=1)
    t = jnp.dot(x, w1en_ref[...], preferred_element_type=f32).astype(bf)
    t = t.reshape(R, K, H) + psb[:, None, :]
    t = _gelu_bf(t).reshape(E_BLK, H)
    t = (jnp.dot(t, w2_ref[...], preferred_element_type=f32).astype(bf)
         + b2_ref[...].astype(bf))
    t = _gelu_bf(t)
    xs = jnp.sum(t.reshape(R, K, H) * ma_ref[0][:, :, None], axis=1)
    msum = jnp.sum(ma_ref[0], axis=1, keepdims=True)
    dh = (jnp.dot(xs.astype(bf), w3_ref[...], preferred_element_type=f32)
          + msum * b3_ref[...]) * (1.0 / SCALE)
    h1 = _ln(hv + dh, n1g_ref[...], n1b_ref[...])
    ff = (jnp.dot(h1.astype(bf), win_ref[...],
                  preferred_element_type=f32).astype(bf)
          + bin_ref[...].astype(bf))
    ff = _gelu_bf(ff)
    d2 = jnp.dot(ff, wout_ref[...], preferred_element_type=f32) + bout_ref[...]
    h2 = _ln(h1 + d2, n2g_ref[...], n2b_ref[...]) * mv_ref[0]
    out_ref[0] = h2
    outb_ref[0] = h2.astype(bf)


def _node_update(h_V, h_E2, g1, mask_attend, mask_V3, wp, nb):
    grid = (nb, L // R)
    full = lambda shape: pl.BlockSpec(shape, lambda b, i: (0,) * len(shape))
    in_specs = [
        pl.BlockSpec((1, R, H), lambda b, i: (b, i, 0)),
        pl.BlockSpec((1, E_BLK, H), lambda b, i: (b, i, 0)),
        pl.BlockSpec((1, E_BLK, H), lambda b, i: (b, i, 0)),
        pl.BlockSpec((1, R, K), lambda b, i: (b, i, 0)),
        pl.BlockSpec((1, R, 1), lambda b, i: (b, i, 0)),
        full((H, H)), full((2 * H, H)), full((1, H)),
        full((H, H)), full((1, H)), full((H, H)), full((1, H)),
        full((1, H)), full((1, H)),
        full((H, 4 * H)), full((1, 4 * H)), full((4 * H, H)), full((1, H)),
        full((1, H)), full((1, H)),
    ]
    out_specs = [
        pl.BlockSpec((1, R, H), lambda b, i: (b, i, 0)),
        pl.BlockSpec((1, R, H), lambda b, i: (b, i, 0)),
    ]
    return pl.pallas_call(
        _node_body,
        grid=grid,
        in_specs=in_specs,
        out_specs=out_specs,
        out_shape=[
            jax.ShapeDtypeStruct((nb, L, H), jnp.float32),
            jax.ShapeDtypeStruct((nb, L, H), jnp.bfloat16),
        ],
        name="tc_node_update",
    )(h_V, h_E2, g1, mask_attend, mask_V3, *wp)


def _edge_body(he_ref, g2_ref, hv2_ref,
               w1s_ref, w1en_ref, b1_ref, w2_ref, b2_ref, w3_ref, b3_ref,
               n3g_ref, n3b_ref, out_ref):
    f32 = jnp.float32
    bf = jnp.bfloat16
    hv2 = hv2_ref[0]
    pre_s = jnp.dot(hv2, w1s_ref[...], preferred_element_type=f32) + b1_ref[...]
    psb = pre_s.astype(bf)
    heb = he_ref[0].astype(bf)
    x = jnp.concatenate([heb, g2_ref[0].astype(bf)], axis=1)
    t = jnp.dot(x, w1en_ref[...], preferred_element_type=f32).astype(bf)
    t = t.reshape(R, K, H) + psb[:, None, :]
    t = _gelu_bf(t).reshape(E_BLK, H)
    t = (jnp.dot(t, w2_ref[...], preferred_element_type=f32).astype(bf)
         + b2_ref[...].astype(bf))
    t = _gelu_bf(t)
    msg = (jnp.dot(t, w3_ref[...], preferred_element_type=f32).astype(bf)
           + b3_ref[...].astype(bf))
    u = heb + msg
    m = jnp.mean(u, axis=-1, keepdims=True)
    c = u - m
    v = jnp.mean(c * c, axis=-1, keepdims=True)
    n = c * lax.rsqrt(v + jnp.bfloat16(1e-5))
    out_ref[0] = (n * n3g_ref[...].astype(bf)
                  + n3b_ref[...].astype(bf)).astype(f32)


def _edge_update(h_E2, g2, hV2b, wp):
    grid = (B, L // R)
    full = lambda shape: pl.BlockSpec(shape, lambda b, i: (0,) * len(shape))
    in_specs = [
        pl.BlockSpec((1, E_BLK, H), lambda b, i: (b, i, 0)),
        pl.BlockSpec((1, E_BLK, H), lambda b, i: (b, i, 0)),
        pl.BlockSpec((1, R, H), lambda b, i: (b, i, 0)),
        full((H, H)), full((2 * H, H)), full((1, H)),
        full((H, H)), full((1, H)), full((H, H)), full((1, H)),
        full((1, H)), full((1, H)),
    ]
    return pl.pallas_call(
        _edge_body,
        grid=grid,
        in_specs=in_specs,
        out_specs=pl.BlockSpec((1, E_BLK, H), lambda b, i: (b, i, 0)),
        out_shape=jax.ShapeDtypeStruct((B, L * K, H), jnp.float32),
        compiler_params=pltpu.CompilerParams(
            vmem_limit_bytes=63 * 1024 * 1024),
        name="tc_edge_update",
    )(h_E2, g2, hV2b, *wp)


def kernel(h_V, h_E, E_idx, mask_V, mask_attend, params):
    p = params
    bf = jnp.bfloat16
    f32 = jnp.float32

    h_E2 = h_E.reshape(B, L * K, H)
    idx_flat = (E_idx.astype(jnp.int32)
                + (jnp.arange(B, dtype=jnp.int32) * L)[:, None, None])
    idx_flat = idx_flat.reshape(TOTAL)
    mask_V3 = mask_V.reshape(B, L, 1)

    def wt(w):
        return jnp.transpose(w).astype(bf)

    def bias(b, n):
        return b.reshape(1, n).astype(f32)

    w1 = jnp.transpose(p['W1_w']).astype(bf)
    wp_node = (
        w1[:H], w1[H:], bias(p['W1_b'], H),
        wt(p['W2_w']), bias(p['W2_b'], H),
        wt(p['W3_w']), bias(p['W3_b'], H),
        bias(p['n1_g'], H), bias(p['n1_b'], H),
        wt(p['Win_w']), bias(p['Win_b'], 4 * H),
        wt(p['Wout_w']), bias(p['Wout_b'], H),
        bias(p['n2_g'], H), bias(p['n2_b'], H),
    )
    w11 = jnp.transpose(p['W11_w']).astype(bf)
    wp_edge = (
        w11[:H], w11[H:], bias(p['W11_b'], H),
        wt(p['W12_w']), bias(p['W12_b'], H),
        wt(p['W13_w']), bias(p['W13_b'], H),
        bias(p['n3_g'], H), bias(p['n3_b'], H),
    )

    g1 = _sc_gather(h_V.reshape(B * L, H), idx_flat, B)
    hV_new, hV_new_b = _node_update(
        h_V, h_E2, g1, mask_attend, mask_V3, wp_node, B)
    g2 = _sc_gather(hV_new.reshape(B * L, H), idx_flat, B)
    hE_new = _edge_update(h_E2, g2, hV_new_b, wp_edge)
    return (hV_new, hE_new.reshape(B, L, K, H))
